# baseline (speedup 1.0000x reference)
import functools

import jax
import jax.numpy as jnp
import numpy as np
from jax import lax
from jax.experimental import pallas as pl
from jax.experimental.pallas import tpu as pltpu
from jax.experimental.pallas import tpu_sc as plsc

N_NODES = 10000
N_EDGES = 160000

NC = 2
NS = 16
LANES = 16

CHUNK = 64
E_PAD = 163840
N_PAD = 10240
ROWS_PER_TILE = N_PAD // NS
RCHUNK = 64
ROW_CHUNKS = ROWS_PER_TILE // RCHUNK
NG = 3
CG = 128
VECS = CG // LANES
SLAB = 16
P0_SLABS = E_PAD // NS // CHUNK // SLAB
P1_SLABS = E_PAD // NC // NS // CHUNK // SLAB
IDXROWS = E_PAD // CHUNK

EB = 2048
NB = 2000
EBLOCKS = E_PAD // EB
LAST_IN_BLOCK = (N_EDGES - 1) // EB


def _edge_kernel(rb0_ref, rb1_ref, sh0_ref, sh1_ref, w10_ref, w11_ref,
                 a0_ref, a1_ref, w20_ref, w21_ref, out_ref):
    b = pl.program_id(0)
    row = b * EB + lax.broadcasted_iota(jnp.int32, (EB, 1), 0)
    mask = row < N_EDGES
    sh0 = sh0_ref[...]
    sh1 = sh1_ref[...]
    h0 = jnp.dot(rb0_ref[...], w10_ref[...],
                 preferred_element_type=jnp.float32)
    s0 = h0 * jax.nn.sigmoid(h0)
    h1 = jnp.dot(rb1_ref[...], w11_ref[...],
                 preferred_element_type=jnp.float32)
    s1 = h1 * jax.nn.sigmoid(h1)
    for g in range(NG):
        t0 = jnp.dot(sh0, a0_ref[g], preferred_element_type=jnp.float32)
        t1 = jnp.dot(sh1, a1_ref[g], preferred_element_type=jnp.float32)
        r0 = jnp.dot(s0, w20_ref[g], preferred_element_type=jnp.float32)
        r1 = jnp.dot(s1, w21_ref[g], preferred_element_type=jnp.float32)
        out_ref[g] = jnp.where(mask, t0 * r0 + t1 * r1, 0.0)


def _edge_values(rb0, rb1, sh0, sh1, W1_0, W1_1, A0g, A1g, W20g, W21g):
    full = lambda s: pl.BlockSpec(s, lambda b: (0,) * len(s))
    clamp = lambda b: (jnp.minimum(b, LAST_IN_BLOCK), 0)
    return pl.pallas_call(
        _edge_kernel,
        grid=(EBLOCKS,),
        in_specs=[
            pl.BlockSpec((EB, 8), clamp),
            pl.BlockSpec((EB, 4), clamp),
            pl.BlockSpec((EB, 1), clamp),
            pl.BlockSpec((EB, 3), clamp),
            full((8, 64)), full((4, 64)),
            full((NG, 1, CG)), full((NG, 3, CG)),
            full((NG, 64, CG)), full((NG, 64, CG)),
        ],
        out_specs=pl.BlockSpec((NG, EB, CG), lambda b: (0, b, 0)),
        out_shape=jax.ShapeDtypeStruct((NG, E_PAD, CG), jnp.float32),
    )(rb0, rb1, sh0, sh1, W1_0, W1_1, A0g, A1g, W20g, W21g)


def _table_kernel(f0_ref, f1_ref, ft0_ref, ft1_ref, out_ref):
    f0 = f0_ref[...]
    f1 = f1_ref[...]
    for g in range(NG):
        out_ref[g] = (
            jnp.dot(f0, ft0_ref[g], preferred_element_type=jnp.float32)
            + jnp.dot(f1, ft1_ref[g], preferred_element_type=jnp.float32))


def _node_table(f0, f1, F0g, F1g):
    full = lambda s: pl.BlockSpec(s, lambda b: (0,) * len(s))
    return pl.pallas_call(
        _table_kernel,
        grid=(N_NODES // NB,),
        in_specs=[
            pl.BlockSpec((NB, 128), lambda b: (b, 0)),
            pl.BlockSpec((NB, 192), lambda b: (b, 0)),
            full((NG, 128, CG)), full((NG, 192, CG)),
        ],
        out_specs=pl.BlockSpec((NG, NB, CG), lambda b: (0, b, 0)),
        out_shape=jax.ShapeDtypeStruct((NG, N_NODES, CG), jnp.float32),
    )(f0, f1, F0g, F1g)


def _sc_body(tab_hbm, ve_hbm, nbr_hbm, ctr_hbm, out_hbm,
             nbr_all, ctr_all, frows, verows, prod, acc, sg, sv, ss):
    c = lax.axis_index("c")
    s = lax.axis_index("s")
    zero = jnp.zeros((LANES,), jnp.float32)
    row0 = s * ROWS_PER_TILE
    m8 = lambda x: pl.multiple_of(x, CHUNK)

    def _gather(k, b):
        pltpu.async_copy(tab_hbm.at[nbr_all.at[k]], frows.at[b], sg.at[b])

    def _ve_load(ebase, k, b):
        pltpu.async_copy(ve_hbm.at[pl.ds(m8(ebase + k * CHUNK), CHUNK)],
                         verows.at[b], sv.at[b])

    def _zero_acc():
        def _zrow(i, _):
            for j in range(VECS):
                verows[0, i, pl.ds(j * LANES, LANES)] = zero
            return 0

        lax.fori_loop(0, RCHUNK, _zrow, 0)
        for k in range(ROW_CHUNKS):
            pltpu.sync_copy(verows.at[0],
                            acc.at[pl.ds(m8(row0 + k * RCHUNK), RCHUNK)])

    def _drain_scatter():
        pltpu.make_async_copy(prod, acc.at[ctr_all.at[0]], ss).wait()

    def _chunk(k, b, drain_prev, issue_next, ebase):
        pltpu.make_async_copy(tab_hbm.at[nbr_all.at[0]],
                              frows.at[b], sg.at[b]).wait()
        pltpu.make_async_copy(ve_hbm.at[pl.ds(m8(ebase), CHUNK)],
                              verows.at[b], sv.at[b]).wait()
        if drain_prev:
            _drain_scatter()

        def _mul(i, _):
            for j in range(VECS):
                sl = pl.ds(j * LANES, LANES)
                prod[i, sl] = frows[b, i, sl] * verows[b, i, sl]
            return 0

        lax.fori_loop(0, CHUNK, _mul, 0)
        pltpu.async_copy(prod, acc.at[ctr_all.at[k]], ss, add=True)
        if issue_next:
            _gather(k + 2, b)
            _ve_load(ebase, k + 2, b)

    def _slab(ibase, cbase, ebase):
        pltpu.sync_copy(nbr_hbm.at[pl.ds(pl.multiple_of(ibase, 8), SLAB)],
                        nbr_all)
        pltpu.sync_copy(ctr_hbm.at[pl.ds(pl.multiple_of(cbase, 8), SLAB)],
                        ctr_all)
        _gather(0, 0)
        _ve_load(ebase, 0, 0)
        _gather(1, 1)
        _ve_load(ebase, 1, 1)
        _chunk(0, 0, False, True, ebase)
        _chunk(1, 1, True, True, ebase)

        def _pair(m, _):
            _chunk(2 * m, 0, True, True, ebase)
            _chunk(2 * m + 1, 1, True, True, ebase)
            return 0

        lax.fori_loop(1, SLAB // 2 - 1, _pair, 0)
        _chunk(SLAB - 2, 0, True, False, ebase)
        _chunk(SLAB - 1, 1, True, False, ebase)
        _drain_scatter()

    def _writeout(obase):
        for k in range(ROW_CHUNKS):
            pltpu.sync_copy(acc.at[pl.ds(m8(row0 + k * RCHUNK), RCHUNK)],
                            out_hbm.at[pl.ds(m8(obase + row0 + k * RCHUNK),
                                             RCHUNK)])

    _zero_acc()
    plsc.subcore_barrier()
    for h in range(P0_SLABS):
        _slab(c * IDXROWS + s * P0_SLABS * SLAB + h * SLAB,
              s * P0_SLABS * SLAB + h * SLAB,
              c * E_PAD + (s * P0_SLABS + h) * SLAB * CHUNK)
    plsc.subcore_barrier()
    _writeout(c * N_PAD)
    _zero_acc()
    plsc.subcore_barrier()
    half_rows = c * (IDXROWS // 2)
    for h in range(P1_SLABS):
        _slab(2 * IDXROWS + half_rows + (s * P1_SLABS + h) * SLAB,
              half_rows + (s * P1_SLABS + h) * SLAB,
              2 * E_PAD + c * (E_PAD // 2) + (s * P1_SLABS + h) * SLAB * CHUNK)
    plsc.subcore_barrier()
    _writeout((2 + c) * N_PAD)


@functools.cache
def _sc_pool_fn():
    return pl.kernel(
        _sc_body,
        out_type=jax.ShapeDtypeStruct((4 * N_PAD, CG), jnp.float32),
        mesh=plsc.VectorSubcoreMesh(core_axis_name="c", subcore_axis_name="s"),
        scratch_types=[
            pltpu.VMEM((SLAB, CHUNK), jnp.int32),
            pltpu.VMEM((SLAB, CHUNK), jnp.int32),
            pltpu.VMEM((2, CHUNK, CG), jnp.float32),
            pltpu.VMEM((2, CHUNK, CG), jnp.float32),
            pltpu.VMEM((CHUNK, CG), jnp.float32),
            pltpu.VMEM_SHARED((N_PAD, CG), jnp.float32),
            pltpu.SemaphoreType.DMA((2,)),
            pltpu.SemaphoreType.DMA((2,)),
            pltpu.SemaphoreType.DMA,
        ],
    )


def _sc_pool(tab3, ve3, nbr3, ctr):
    return _sc_pool_fn()(tab3, ve3, nbr3, ctr)


def _out_kernel(p_ref, f0_ref, f1_ref, g_ref, out0_ref, out1_ref):
    acc = jnp.dot(p_ref[0], g_ref[0], preferred_element_type=jnp.float32)
    acc = acc + jnp.dot(p_ref[1], g_ref[1],
                        preferred_element_type=jnp.float32)
    acc = acc + jnp.dot(p_ref[2] + p_ref[3], g_ref[2],
                        preferred_element_type=jnp.float32)
    out0_ref[...] = f0_ref[...] + acc[:, 0:128]
    out1_ref[...] = f1_ref[...] + acc[:, 128:320]


def _couple_out(pooled, f0, f1, Gg):
    full = lambda s: pl.BlockSpec(s, lambda b: (0,) * len(s))
    return pl.pallas_call(
        _out_kernel,
        grid=(N_NODES // NB,),
        in_specs=[
            pl.BlockSpec((4, NB, CG), lambda b: (0, b, 0)),
            pl.BlockSpec((NB, 128), lambda b: (b, 0)),
            pl.BlockSpec((NB, 192), lambda b: (b, 0)),
            full((NG, CG, 320)),
        ],
        out_specs=[
            pl.BlockSpec((NB, 128), lambda b: (b, 0)),
            pl.BlockSpec((NB, 192), lambda b: (b, 0)),
        ],
        out_shape=[
            jax.ShapeDtypeStruct((N_NODES, 128), jnp.float32),
            jax.ShapeDtypeStruct((N_NODES, 192), jnp.float32),
        ],
    )(pooled, f0, f1, Gg)


def _fold_constants(W2_0, W2_1, Wlin_0, Wlin_1, U_0, U_1):
    z64 = jnp.zeros((1, 64), jnp.float32)
    one = jnp.ones((1, 64), jnp.float32)
    A0 = jnp.concatenate(
        [U_0[0:1, 0:1] * one] +
        [U_1[i:i + 1, 0:1] * one for i in range(4)] + [z64], axis=1)
    A1 = jnp.concatenate(
        [jnp.zeros((3, 64), jnp.float32)] +
        [U_1[i, 1:4].reshape(3, 1) * jnp.ones((3, 64), jnp.float32)
         for i in range(4)] + [jnp.zeros((3, 64), jnp.float32)], axis=1)
    zw = jnp.zeros((64, 64), jnp.float32)
    W20f = jnp.concatenate(
        [W2_0[:, 64:128]] + [W2_0[:, 0:64]] * 4 + [zw], axis=1)
    W21f = jnp.concatenate([zw] + [W2_1] * 4 + [zw], axis=1)

    eye = np.eye(64, dtype=np.float32)
    blk = lambda r, c, rows, cols: np.pad(
        eye, ((r * 64, rows - 64 * (r + 1)), (c * 64, cols - 64 * (c + 1))))
    F0 = U_0[0, 0] * jnp.asarray(blk(1, 0, 128, 384))
    for i in range(4):
        F0 = F0 + U_1[i, 0] * jnp.asarray(blk(0, 1 + i, 128, 384))
    F1 = jnp.zeros((192, 384), jnp.float32)
    for i in range(4):
        for j in range(1, 4):
            F1 = F1 + U_1[i, j] * jnp.asarray(blk(j - 1, 1 + i, 192, 384))

    B0 = U_0[0, 0] * jnp.asarray(blk(0, 0, 384, 128))
    for j in range(4):
        B0 = B0 + U_1[j, 0] * jnp.asarray(blk(1 + j, 1, 384, 128))
    G = jnp.dot(B0, Wlin_0)
    for m in range(1, 4):
        Bm = jnp.zeros((384, 64), jnp.float32)
        for j in range(4):
            Bm = Bm + U_1[j, m] * jnp.asarray(blk(1 + j, 0, 384, 64))
        G = jnp.concatenate([G, jnp.dot(Bm, Wlin_1)], axis=1)

    group = lambda x: x.reshape(x.shape[0], NG, CG).transpose(1, 0, 2)
    return (group(A0), group(A1), group(W20f), group(W21f),
            group(F0), group(F1), G.reshape(NG, CG, 320))


def kernel(radial_basis_0, radial_basis_1, spherical_harmonics_0,
           spherical_harmonics_1, centers, neighbors, features_0, features_1,
           W1_0, W2_0, W1_1, W2_1, Wlin_0, Wlin_1, U_0, U_1):
    pad = E_PAD - N_EDGES
    ctr = jnp.pad(centers.astype(jnp.int32), (0, pad))
    nbr = jnp.pad(neighbors.astype(jnp.int32), (0, pad))
    nbr3 = jnp.concatenate([nbr + g * N_NODES for g in range(NG)])

    f0 = features_0.reshape(N_NODES, 128)
    f1 = features_1.reshape(N_NODES, 192)

    (A0g, A1g, W20g, W21g, F0g, F1g, Gg) = _fold_constants(
        W2_0, W2_1, Wlin_0, Wlin_1, U_0, U_1)

    ve = _edge_values(radial_basis_0, radial_basis_1, spherical_harmonics_0,
                      spherical_harmonics_1, W1_0, W1_1, A0g, A1g, W20g, W21g)
    tab = _node_table(f0, f1, F0g, F1g)
    pooled = _sc_pool(tab.reshape(NG * N_NODES, CG),
                      ve.reshape(NG * E_PAD, CG),
                      nbr3.reshape(-1, CHUNK), ctr.reshape(-1, CHUNK))
    pooled = pooled.reshape(4, N_PAD, CG)
    out0, out1 = _couple_out(pooled, f0, f1, Gg)
    return (out0.reshape(N_NODES, 1, 128), out1.reshape(N_NODES, 3, 64))


# speedup vs baseline: 1.0136x; 1.0136x over previous
import functools

import jax
import jax.numpy as jnp
import numpy as np
from jax import lax
from jax.experimental import pallas as pl
from jax.experimental.pallas import tpu as pltpu
from jax.experimental.pallas import tpu_sc as plsc

N_NODES = 10000
N_EDGES = 160000

NC = 2
NS = 16
LANES = 16

CHUNK = 64
E_PAD = 163840
N_PAD = 10240
ROWS_PER_TILE = N_PAD // NS
RCHUNK = 64
ROW_CHUNKS = ROWS_PER_TILE // RCHUNK
NG = 3
CG = 128
VECS = CG // LANES
SLAB = 40
P0_SLABS = E_PAD // NS // CHUNK // SLAB
P1_SLABS = E_PAD // NC // NS // CHUNK // SLAB
IDXROWS = E_PAD // CHUNK

EB = 2048
NB = 2000
EBLOCKS = E_PAD // EB
LAST_IN_BLOCK = (N_EDGES - 1) // EB


def _edge_kernel(rb0_ref, rb1_ref, sh0_ref, sh1_ref, w10_ref, w11_ref,
                 a0_ref, a1_ref, w20_ref, w21_ref, out_ref):
    b = pl.program_id(0)
    row = b * EB + lax.broadcasted_iota(jnp.int32, (EB, 1), 0)
    mask = row < N_EDGES
    sh0 = sh0_ref[...]
    sh1 = sh1_ref[...]
    h0 = jnp.dot(rb0_ref[...], w10_ref[...],
                 preferred_element_type=jnp.float32)
    s0 = h0 * jax.nn.sigmoid(h0)
    h1 = jnp.dot(rb1_ref[...], w11_ref[...],
                 preferred_element_type=jnp.float32)
    s1 = h1 * jax.nn.sigmoid(h1)
    for g in range(NG):
        t0 = jnp.dot(sh0, a0_ref[g], preferred_element_type=jnp.float32)
        t1 = jnp.dot(sh1, a1_ref[g], preferred_element_type=jnp.float32)
        r0 = jnp.dot(s0, w20_ref[g], preferred_element_type=jnp.float32)
        r1 = jnp.dot(s1, w21_ref[g], preferred_element_type=jnp.float32)
        out_ref[g] = jnp.where(mask, t0 * r0 + t1 * r1, 0.0)


def _edge_values(rb0, rb1, sh0, sh1, W1_0, W1_1, A0g, A1g, W20g, W21g):
    full = lambda s: pl.BlockSpec(s, lambda b: (0,) * len(s))
    clamp = lambda b: (jnp.minimum(b, LAST_IN_BLOCK), 0)
    return pl.pallas_call(
        _edge_kernel,
        grid=(EBLOCKS,),
        in_specs=[
            pl.BlockSpec((EB, 8), clamp),
            pl.BlockSpec((EB, 4), clamp),
            pl.BlockSpec((EB, 1), clamp),
            pl.BlockSpec((EB, 3), clamp),
            full((8, 64)), full((4, 64)),
            full((NG, 1, CG)), full((NG, 3, CG)),
            full((NG, 64, CG)), full((NG, 64, CG)),
        ],
        out_specs=pl.BlockSpec((NG, EB, CG), lambda b: (0, b, 0)),
        out_shape=jax.ShapeDtypeStruct((NG, E_PAD, CG), jnp.float32),
    )(rb0, rb1, sh0, sh1, W1_0, W1_1, A0g, A1g, W20g, W21g)


def _table_kernel(f0_ref, f1_ref, ft0_ref, ft1_ref, out_ref):
    f0 = f0_ref[...]
    f1 = f1_ref[...]
    for g in range(NG):
        out_ref[g] = (
            jnp.dot(f0, ft0_ref[g], preferred_element_type=jnp.float32)
            + jnp.dot(f1, ft1_ref[g], preferred_element_type=jnp.float32))


def _node_table(f0, f1, F0g, F1g):
    full = lambda s: pl.BlockSpec(s, lambda b: (0,) * len(s))
    return pl.pallas_call(
        _table_kernel,
        grid=(N_NODES // NB,),
        in_specs=[
            pl.BlockSpec((NB, 128), lambda b: (b, 0)),
            pl.BlockSpec((NB, 192), lambda b: (b, 0)),
            full((NG, 128, CG)), full((NG, 192, CG)),
        ],
        out_specs=pl.BlockSpec((NG, NB, CG), lambda b: (0, b, 0)),
        out_shape=jax.ShapeDtypeStruct((NG, N_NODES, CG), jnp.float32),
    )(f0, f1, F0g, F1g)


def _sc_body(tab_hbm, ve_hbm, nbr_hbm, ctr_hbm, out_hbm,
             nbr_all, ctr_all, frows, verows, acc, sg, sv, ss):
    c = lax.axis_index("c")
    s = lax.axis_index("s")
    zero = jnp.zeros((LANES,), jnp.float32)
    row0 = s * ROWS_PER_TILE
    m8 = lambda x: pl.multiple_of(x, CHUNK)

    def _gather(k, b):
        pltpu.async_copy(tab_hbm.at[nbr_all.at[k]], frows.at[b], sg.at[b])

    def _ve_load(ebase, k, b):
        pltpu.async_copy(ve_hbm.at[pl.ds(m8(ebase + k * CHUNK), CHUNK)],
                         verows.at[b], sv.at[b])

    def _zero_acc():
        def _zrow(i, _):
            for j in range(VECS):
                verows[0, i, pl.ds(j * LANES, LANES)] = zero
            return 0

        lax.fori_loop(0, RCHUNK, _zrow, 0)
        for k in range(ROW_CHUNKS):
            pltpu.sync_copy(verows.at[0],
                            acc.at[pl.ds(m8(row0 + k * RCHUNK), RCHUNK)])

    def _chunk(k, b, issue_next, ebase):
        pltpu.make_async_copy(tab_hbm.at[nbr_all.at[0]],
                              frows.at[b], sg.at[b]).wait()
        pltpu.make_async_copy(ve_hbm.at[pl.ds(m8(ebase), CHUNK)],
                              verows.at[b], sv.at[b]).wait()

        def _mul(i, _):
            for j in range(VECS):
                sl = pl.ds(j * LANES, LANES)
                frows[b, i, sl] = frows[b, i, sl] * verows[b, i, sl]
            return 0

        lax.fori_loop(0, CHUNK, _mul, 0)
        pltpu.async_copy(frows.at[b], acc.at[ctr_all.at[k]], ss.at[b],
                         add=True)
        pltpu.make_async_copy(frows.at[b], acc.at[ctr_all.at[0]],
                              ss.at[b]).wait()
        if issue_next:
            _gather(k + 2, b)
            _ve_load(ebase, k + 2, b)

    def _slab(ibase, cbase, ebase):
        pltpu.sync_copy(nbr_hbm.at[pl.ds(pl.multiple_of(ibase, 8), SLAB)],
                        nbr_all)
        pltpu.sync_copy(ctr_hbm.at[pl.ds(pl.multiple_of(cbase, 8), SLAB)],
                        ctr_all)
        _gather(0, 0)
        _ve_load(ebase, 0, 0)
        _gather(1, 1)
        _ve_load(ebase, 1, 1)

        def _pair(m, _):
            _chunk(2 * m, 0, True, ebase)
            _chunk(2 * m + 1, 1, True, ebase)
            return 0

        lax.fori_loop(0, SLAB // 2 - 1, _pair, 0)
        _chunk(SLAB - 2, 0, False, ebase)
        _chunk(SLAB - 1, 1, False, ebase)

    def _writeout(obase):
        for k in range(ROW_CHUNKS):
            pltpu.sync_copy(acc.at[pl.ds(m8(row0 + k * RCHUNK), RCHUNK)],
                            out_hbm.at[pl.ds(m8(obase + row0 + k * RCHUNK),
                                             RCHUNK)])

    _zero_acc()
    plsc.subcore_barrier()
    for h in range(P0_SLABS):
        _slab(c * IDXROWS + s * P0_SLABS * SLAB + h * SLAB,
              s * P0_SLABS * SLAB + h * SLAB,
              c * E_PAD + (s * P0_SLABS + h) * SLAB * CHUNK)
    plsc.subcore_barrier()
    _writeout(c * N_PAD)
    _zero_acc()
    plsc.subcore_barrier()
    half_rows = c * (IDXROWS // 2)
    for h in range(P1_SLABS):
        _slab(2 * IDXROWS + half_rows + (s * P1_SLABS + h) * SLAB,
              half_rows + (s * P1_SLABS + h) * SLAB,
              2 * E_PAD + c * (E_PAD // 2) + (s * P1_SLABS + h) * SLAB * CHUNK)
    plsc.subcore_barrier()
    _writeout((2 + c) * N_PAD)


@functools.cache
def _sc_pool_fn():
    return pl.kernel(
        _sc_body,
        out_type=jax.ShapeDtypeStruct((4 * N_PAD, CG), jnp.float32),
        mesh=plsc.VectorSubcoreMesh(core_axis_name="c", subcore_axis_name="s"),
        scratch_types=[
            pltpu.VMEM((SLAB, CHUNK), jnp.int32),
            pltpu.VMEM((SLAB, CHUNK), jnp.int32),
            pltpu.VMEM((2, CHUNK, CG), jnp.float32),
            pltpu.VMEM((2, CHUNK, CG), jnp.float32),
            pltpu.VMEM_SHARED((N_PAD, CG), jnp.float32),
            pltpu.SemaphoreType.DMA((2,)),
            pltpu.SemaphoreType.DMA((2,)),
            pltpu.SemaphoreType.DMA((2,)),
        ],
    )


def _sc_pool(tab3, ve3, nbr3, ctr):
    return _sc_pool_fn()(tab3, ve3, nbr3, ctr)


def _out_kernel(p_ref, f0_ref, f1_ref, g_ref, out0_ref, out1_ref):
    acc = jnp.dot(p_ref[0], g_ref[0], preferred_element_type=jnp.float32)
    acc = acc + jnp.dot(p_ref[1], g_ref[1],
                        preferred_element_type=jnp.float32)
    acc = acc + jnp.dot(p_ref[2] + p_ref[3], g_ref[2],
                        preferred_element_type=jnp.float32)
    out0_ref[...] = f0_ref[...] + acc[:, 0:128]
    out1_ref[...] = f1_ref[...] + acc[:, 128:320]


def _couple_out(pooled, f0, f1, Gg):
    full = lambda s: pl.BlockSpec(s, lambda b: (0,) * len(s))
    return pl.pallas_call(
        _out_kernel,
        grid=(N_NODES // NB,),
        in_specs=[
            pl.BlockSpec((4, NB, CG), lambda b: (0, b, 0)),
            pl.BlockSpec((NB, 128), lambda b: (b, 0)),
            pl.BlockSpec((NB, 192), lambda b: (b, 0)),
            full((NG, CG, 320)),
        ],
        out_specs=[
            pl.BlockSpec((NB, 128), lambda b: (b, 0)),
            pl.BlockSpec((NB, 192), lambda b: (b, 0)),
        ],
        out_shape=[
            jax.ShapeDtypeStruct((N_NODES, 128), jnp.float32),
            jax.ShapeDtypeStruct((N_NODES, 192), jnp.float32),
        ],
    )(pooled, f0, f1, Gg)


def _fold_constants(W2_0, W2_1, Wlin_0, Wlin_1, U_0, U_1):
    z64 = jnp.zeros((1, 64), jnp.float32)
    one = jnp.ones((1, 64), jnp.float32)
    A0 = jnp.concatenate(
        [U_0[0:1, 0:1] * one] +
        [U_1[i:i + 1, 0:1] * one for i in range(4)] + [z64], axis=1)
    A1 = jnp.concatenate(
        [jnp.zeros((3, 64), jnp.float32)] +
        [U_1[i, 1:4].reshape(3, 1) * jnp.ones((3, 64), jnp.float32)
         for i in range(4)] + [jnp.zeros((3, 64), jnp.float32)], axis=1)
    zw = jnp.zeros((64, 64), jnp.float32)
    W20f = jnp.concatenate(
        [W2_0[:, 64:128]] + [W2_0[:, 0:64]] * 4 + [zw], axis=1)
    W21f = jnp.concatenate([zw] + [W2_1] * 4 + [zw], axis=1)

    eye = np.eye(64, dtype=np.float32)
    blk = lambda r, c, rows, cols: np.pad(
        eye, ((r * 64, rows - 64 * (r + 1)), (c * 64, cols - 64 * (c + 1))))
    F0 = U_0[0, 0] * jnp.asarray(blk(1, 0, 128, 384))
    for i in range(4):
        F0 = F0 + U_1[i, 0] * jnp.asarray(blk(0, 1 + i, 128, 384))
    F1 = jnp.zeros((192, 384), jnp.float32)
    for i in range(4):
        for j in range(1, 4):
            F1 = F1 + U_1[i, j] * jnp.asarray(blk(j - 1, 1 + i, 192, 384))

    B0 = U_0[0, 0] * jnp.asarray(blk(0, 0, 384, 128))
    for j in range(4):
        B0 = B0 + U_1[j, 0] * jnp.asarray(blk(1 + j, 1, 384, 128))
    G = jnp.dot(B0, Wlin_0)
    for m in range(1, 4):
        Bm = jnp.zeros((384, 64), jnp.float32)
        for j in range(4):
            Bm = Bm + U_1[j, m] * jnp.asarray(blk(1 + j, 0, 384, 64))
        G = jnp.concatenate([G, jnp.dot(Bm, Wlin_1)], axis=1)

    group = lambda x: x.reshape(x.shape[0], NG, CG).transpose(1, 0, 2)
    return (group(A0), group(A1), group(W20f), group(W21f),
            group(F0), group(F1), G.reshape(NG, CG, 320))


def kernel(radial_basis_0, radial_basis_1, spherical_harmonics_0,
           spherical_harmonics_1, centers, neighbors, features_0, features_1,
           W1_0, W2_0, W1_1, W2_1, Wlin_0, Wlin_1, U_0, U_1):
    pad = E_PAD - N_EDGES
    ctr = jnp.pad(centers.astype(jnp.int32), (0, pad))
    nbr = jnp.pad(neighbors.astype(jnp.int32), (0, pad))
    nbr3 = jnp.concatenate([nbr + g * N_NODES for g in range(NG)])

    f0 = features_0.reshape(N_NODES, 128)
    f1 = features_1.reshape(N_NODES, 192)

    (A0g, A1g, W20g, W21g, F0g, F1g, Gg) = _fold_constants(
        W2_0, W2_1, Wlin_0, Wlin_1, U_0, U_1)

    ve = _edge_values(radial_basis_0, radial_basis_1, spherical_harmonics_0,
                      spherical_harmonics_1, W1_0, W1_1, A0g, A1g, W20g, W21g)
    tab = _node_table(f0, f1, F0g, F1g)
    pooled = _sc_pool(tab.reshape(NG * N_NODES, CG),
                      ve.reshape(NG * E_PAD, CG),
                      nbr3.reshape(-1, CHUNK), ctr.reshape(-1, CHUNK))
    pooled = pooled.reshape(4, N_PAD, CG)
    out0, out1 = _couple_out(pooled, f0, f1, Gg)
    return (out0.reshape(N_NODES, 1, 128), out1.reshape(N_NODES, 3, 64))
